# Initial kernel scaffold; baseline (speedup 1.0000x reference)
import jax
import jax.numpy as jnp
from jax.experimental import pallas as pl


def kernel(logits, target, node_batch, edge_index):
    raise NotImplementedError("write your pallas kernel here")


# speedup vs baseline: 143.9112x; 143.9112x over previous
import jax
import jax.numpy as jnp
from jax import lax
from jax.experimental import pallas as pl
from jax.experimental.pallas import tpu as pltpu
from jax.experimental.pallas import tpu_sc as plsc

N_NODES = 100000
N_EDGES = 3200000
N_BINS = 64

NUM_CORES = 2
NUM_SUBCORES = 16
NW = NUM_CORES * NUM_SUBCORES
EDGES_PER_TILE = N_EDGES // NW
CHUNK = 2000
NCHUNK = EDGES_PER_TILE // CHUNK
VREGS_PER_CHUNK = CHUNK // 16

_LOG1P_C = (
    2.554673020642217e-07,
    0.9999670809438458,
    -0.49928504912221533,
    0.3272257149716124,
    -0.22316586411340317,
    0.1308334279755053,
    -0.05243753706143922,
    0.010009289616292891,
)


def _log1p_poly(u):
    p = jnp.full((16,), _LOG1P_C[-1], dtype=jnp.float32)
    for c in _LOG1P_C[-2::-1]:
        p = p * u + jnp.float32(c)
    return p


def _sc_body(lg_hbm, tg_hbm, ei_hbm, nb_hbm, outl_hbm, outc_hbm,
             nb_v, idx_v0, idx_v1, lg_v0, lg_v1, tg_v0, tg_v1, acc_l, acc_c,
             sem0, sem1):
    w = lax.axis_index("c") * NUM_SUBCORES + lax.axis_index("s")
    base = w * EDGES_PER_TILE
    sems = (sem0, sem1)
    idx_v = (idx_v0, idx_v1)
    lg_v = (lg_v0, lg_v1)
    tg_v = (tg_v0, tg_v1)

    pltpu.sync_copy(nb_hbm, nb_v)

    zeros16 = jnp.zeros((16,), jnp.float32)
    ones16 = jnp.ones((16,), jnp.float32)
    lane64 = lax.iota(jnp.int32, 16) * N_BINS

    @pl.loop(0, N_BINS)
    def _zero(i):
        acc_l[pl.ds(i * 16, 16)] = zeros16
        acc_c[pl.ds(i * 16, 16)] = zeros16

    def _start(c, b):
        off = base + c * CHUNK
        pltpu.async_copy(lg_hbm.at[pl.ds(off, CHUNK)], lg_v[b], sems[b])
        pltpu.async_copy(tg_hbm.at[pl.ds(off, CHUNK)], tg_v[b], sems[b])
        pltpu.async_copy(ei_hbm.at[pl.ds(off, CHUNK)], idx_v[b], sems[b])

    def _wait(c, b):
        off = base + c * CHUNK
        pltpu.make_async_copy(lg_hbm.at[pl.ds(off, CHUNK)], lg_v[b], sems[b]).wait()
        pltpu.make_async_copy(tg_hbm.at[pl.ds(off, CHUNK)], tg_v[b], sems[b]).wait()
        pltpu.make_async_copy(ei_hbm.at[pl.ds(off, CHUNK)], idx_v[b], sems[b]).wait()

    def _compute(b):
        @pl.loop(0, VREGS_PER_CHUNK)
        def _inner(i):
            sl = pl.ds(i * 16, 16)
            src = idx_v[b][sl]
            g = plsc.load_gather(nb_v, [src])
            addr = lane64 + g
            l = lg_v[b][sl]
            t = tg_v[b][sl]
            u = jnp.exp(-jnp.abs(l))
            raw = jnp.maximum(l, 0.0) - l * t + _log1p_poly(u)
            plsc.addupdate_scatter(acc_l, [addr], raw)
            plsc.addupdate_scatter(acc_c, [addr], ones16)

    _start(0, 0)
    _start(1, 1)

    @pl.loop(0, (NCHUNK - 2) // 2)
    def _ring(gi):
        for b in range(2):
            c = gi * 2 + b
            _wait(c, b)
            _start(c + 2, b)
            _compute(b)

    for b in range(2):
        _wait(NCHUNK - 2 + b, b)
        _compute(b)

    half = N_BINS * 16 // 2
    while half >= N_BINS:
        for o in range(0, half, 16):
            sl_lo = pl.ds(o, 16)
            sl_hi = pl.ds(o + half, 16)
            acc_l[sl_lo] = acc_l[sl_lo] + acc_l[sl_hi]
            acc_c[sl_lo] = acc_c[sl_lo] + acc_c[sl_hi]
        half //= 2

    pltpu.sync_copy(acc_l.at[pl.ds(0, N_BINS)], outl_hbm.at[pl.ds(w * N_BINS, N_BINS)])
    pltpu.sync_copy(acc_c.at[pl.ds(0, N_BINS)], outc_hbm.at[pl.ds(w * N_BINS, N_BINS)])


_sc_call = pl.kernel(
    _sc_body,
    out_type=[
        jax.ShapeDtypeStruct((NW * N_BINS,), jnp.float32),
        jax.ShapeDtypeStruct((NW * N_BINS,), jnp.float32),
    ],
    mesh=plsc.VectorSubcoreMesh(core_axis_name="c", subcore_axis_name="s"),
    compiler_params=pltpu.CompilerParams(needs_layout_passes=False),
    scratch_types=[
        pltpu.VMEM((N_NODES,), jnp.int32),
        pltpu.VMEM((CHUNK,), jnp.int32),
        pltpu.VMEM((CHUNK,), jnp.int32),
        pltpu.VMEM((CHUNK,), jnp.float32),
        pltpu.VMEM((CHUNK,), jnp.float32),
        pltpu.VMEM((CHUNK,), jnp.float32),
        pltpu.VMEM((CHUNK,), jnp.float32),
        pltpu.VMEM((N_BINS * 16,), jnp.float32),
        pltpu.VMEM((N_BINS * 16,), jnp.float32),
        pltpu.SemaphoreType.DMA,
        pltpu.SemaphoreType.DMA,
    ],
)


def _fin_body(l_ref, c_ref, n_ref, o_ref):
    gl = jnp.sum(l_ref[...], axis=0, keepdims=True)
    gc = jnp.sum(c_ref[...], axis=0, keepdims=True)
    ratio = gl / jnp.maximum(gc, 1.0)
    o_ref[...] = jnp.sum(ratio, keepdims=True) / n_ref[...]


_fin_call = pl.pallas_call(
    _fin_body,
    out_shape=jax.ShapeDtypeStruct((1, 1), jnp.float32),
)


def kernel(logits, target, node_batch, edge_index):
    logits = logits.astype(jnp.float32)
    target = target.astype(jnp.float32)
    nb = node_batch.astype(jnp.int32)
    ei = edge_index.astype(jnp.int32).reshape(-1)
    outl, outc = _sc_call(logits, target, ei, nb)
    outl = outl.reshape(NW, N_BINS)
    outc = outc.reshape(NW, N_BINS)
    n_graphs = (nb[-1].astype(jnp.float32) + 1.0).reshape(1, 1)
    return _fin_call(outl, outc, n_graphs)[0, 0]


# speedup vs baseline: 151.9762x; 1.0560x over previous
import jax
import jax.numpy as jnp
from jax import lax
from jax.experimental import pallas as pl
from jax.experimental.pallas import tpu as pltpu
from jax.experimental.pallas import tpu_sc as plsc

N_NODES = 100000
N_EDGES = 3200000
N_BINS = 64

NUM_CORES = 2
NUM_SUBCORES = 16
NW = NUM_CORES * NUM_SUBCORES
EDGES_PER_TILE = N_EDGES // NW
CHUNK = 2000
NCHUNK = EDGES_PER_TILE // CHUNK
VREGS_PER_CHUNK = CHUNK // 16

_LOG1P_C = (
    1.6936626588279148e-06,
    0.9998325947817008,
    -0.4972033312209412,
    0.31504127991169084,
    -0.18901954822862585,
    0.08152317762235059,
    -0.017029610590701707,
)


def _log1p_poly(u):
    p = jnp.full((16,), _LOG1P_C[-1], dtype=jnp.float32)
    for c in _LOG1P_C[-2::-1]:
        p = p * u + jnp.float32(c)
    return p


def _sc_body(lg_hbm, tg_hbm, ei_hbm, nb_hbm, outl_hbm, outc_hbm,
             nb_v, idx_v0, idx_v1, lg_v0, lg_v1, tg_v0, tg_v1, acc_l, acc_c,
             sem0, sem1):
    w = lax.axis_index("c") * NUM_SUBCORES + lax.axis_index("s")
    base = w * EDGES_PER_TILE
    sems = (sem0, sem1)
    idx_v = (idx_v0, idx_v1)
    lg_v = (lg_v0, lg_v1)
    tg_v = (tg_v0, tg_v1)

    pltpu.sync_copy(nb_hbm, nb_v)

    zeros16 = jnp.zeros((16,), jnp.float32)
    ones16 = jnp.ones((16,), jnp.float32)
    lane64 = lax.iota(jnp.int32, 16) * N_BINS

    @pl.loop(0, N_BINS)
    def _zero(i):
        acc_l[pl.ds(i * 16, 16)] = zeros16
        acc_c[pl.ds(i * 16, 16)] = zeros16

    def _start(c, b):
        off = base + c * CHUNK
        pltpu.async_copy(lg_hbm.at[pl.ds(off, CHUNK)], lg_v[b], sems[b])
        pltpu.async_copy(tg_hbm.at[pl.ds(off, CHUNK)], tg_v[b], sems[b])
        pltpu.async_copy(ei_hbm.at[pl.ds(off, CHUNK)], idx_v[b], sems[b])

    def _wait(c, b):
        off = base + c * CHUNK
        pltpu.make_async_copy(lg_hbm.at[pl.ds(off, CHUNK)], lg_v[b], sems[b]).wait()
        pltpu.make_async_copy(tg_hbm.at[pl.ds(off, CHUNK)], tg_v[b], sems[b]).wait()
        pltpu.make_async_copy(ei_hbm.at[pl.ds(off, CHUNK)], idx_v[b], sems[b]).wait()

    def _compute(b):
        @pl.loop(0, VREGS_PER_CHUNK, unroll=5)
        def _inner(i):
            sl = pl.ds(i * 16, 16)
            src = idx_v[b][sl]
            g = plsc.load_gather(nb_v, [src])
            addr = lane64 + g
            l = lg_v[b][sl]
            t = tg_v[b][sl]
            u = jnp.exp(-jnp.abs(l))
            raw = jnp.maximum(l, 0.0) - l * t + _log1p_poly(u)
            plsc.addupdate_scatter(acc_l, [addr], raw)
            plsc.addupdate_scatter(acc_c, [addr], ones16)

    _start(0, 0)
    _start(1, 1)

    @pl.loop(0, (NCHUNK - 2) // 2)
    def _ring(gi):
        for b in range(2):
            c = gi * 2 + b
            _wait(c, b)
            _start(c + 2, b)
            _compute(b)

    for b in range(2):
        _wait(NCHUNK - 2 + b, b)
        _compute(b)

    half = N_BINS * 16 // 2
    while half >= N_BINS:
        for o in range(0, half, 16):
            sl_lo = pl.ds(o, 16)
            sl_hi = pl.ds(o + half, 16)
            acc_l[sl_lo] = acc_l[sl_lo] + acc_l[sl_hi]
            acc_c[sl_lo] = acc_c[sl_lo] + acc_c[sl_hi]
        half //= 2

    pltpu.sync_copy(acc_l.at[pl.ds(0, N_BINS)], outl_hbm.at[pl.ds(w * N_BINS, N_BINS)])
    pltpu.sync_copy(acc_c.at[pl.ds(0, N_BINS)], outc_hbm.at[pl.ds(w * N_BINS, N_BINS)])


_sc_call = pl.kernel(
    _sc_body,
    out_type=[
        jax.ShapeDtypeStruct((NW * N_BINS,), jnp.float32),
        jax.ShapeDtypeStruct((NW * N_BINS,), jnp.float32),
    ],
    mesh=plsc.VectorSubcoreMesh(core_axis_name="c", subcore_axis_name="s"),
    compiler_params=pltpu.CompilerParams(needs_layout_passes=False),
    scratch_types=[
        pltpu.VMEM((N_NODES,), jnp.int32),
        pltpu.VMEM((CHUNK,), jnp.int32),
        pltpu.VMEM((CHUNK,), jnp.int32),
        pltpu.VMEM((CHUNK,), jnp.float32),
        pltpu.VMEM((CHUNK,), jnp.float32),
        pltpu.VMEM((CHUNK,), jnp.float32),
        pltpu.VMEM((CHUNK,), jnp.float32),
        pltpu.VMEM((N_BINS * 16,), jnp.float32),
        pltpu.VMEM((N_BINS * 16,), jnp.float32),
        pltpu.SemaphoreType.DMA,
        pltpu.SemaphoreType.DMA,
    ],
)


def _fin_body(l_ref, c_ref, n_ref, o_ref):
    gl = jnp.sum(l_ref[...], axis=0, keepdims=True)
    gc = jnp.sum(c_ref[...], axis=0, keepdims=True)
    ratio = gl / jnp.maximum(gc, 1.0)
    o_ref[...] = jnp.sum(ratio, keepdims=True) / n_ref[...]


_fin_call = pl.pallas_call(
    _fin_body,
    out_shape=jax.ShapeDtypeStruct((1, 1), jnp.float32),
)


def kernel(logits, target, node_batch, edge_index):
    logits = logits.astype(jnp.float32)
    target = target.astype(jnp.float32)
    nb = node_batch.astype(jnp.int32)
    ei = edge_index.astype(jnp.int32).reshape(-1)
    outl, outc = _sc_call(logits, target, ei, nb)
    outl = outl.reshape(NW, N_BINS)
    outc = outc.reshape(NW, N_BINS)
    n_graphs = (nb[-1].astype(jnp.float32) + 1.0).reshape(1, 1)
    return _fin_call(outl, outc, n_graphs)[0, 0]


# speedup vs baseline: 398.8399x; 2.6244x over previous
import jax
import jax.numpy as jnp
from jax import lax
from jax.experimental import pallas as pl
from jax.experimental.pallas import tpu as pltpu
from jax.experimental.pallas import tpu_sc as plsc

N_NODES = 100000
N_EDGES = 3200000
N_BINS = 64

NUM_CORES = 2
NUM_SUBCORES = 16
NW = NUM_CORES * NUM_SUBCORES
EDGES_PER_TILE = N_EDGES // NW
CHUNK = 2000
NCHUNK = EDGES_PER_TILE // CHUNK
VREGS_PER_CHUNK = CHUNK // 16

_LOG1P_C = (
    1.6936626588279148e-06,
    0.9998325947817008,
    -0.4972033312209412,
    0.31504127991169084,
    -0.18901954822862585,
    0.08152317762235059,
    -0.017029610590701707,
)


def _log1p_poly(u):
    p = jnp.full((16,), _LOG1P_C[-1], dtype=jnp.float32)
    for c in _LOG1P_C[-2::-1]:
        p = p * u + jnp.float32(c)
    return p


def _sc_body(lg_hbm, tg_hbm, ei_hbm, nb_hbm, outl_hbm, outc_hbm,
             nb_v, idx_v0, idx_v1, lg_v0, lg_v1, tg_v0, tg_v1, acc_l, acc_c,
             sem0, sem1):
    w = lax.axis_index("c") * NUM_SUBCORES + lax.axis_index("s")
    base = w * EDGES_PER_TILE
    sems = (sem0, sem1)
    idx_v = (idx_v0, idx_v1)
    lg_v = (lg_v0, lg_v1)
    tg_v = (tg_v0, tg_v1)

    pltpu.sync_copy(nb_hbm, nb_v)

    zeros16 = jnp.zeros((16,), jnp.float32)
    ones16 = jnp.ones((16,), jnp.float32)
    lane64 = lax.iota(jnp.int32, 16) * N_BINS

    @pl.loop(0, N_BINS)
    def _zero(i):
        acc_l[pl.ds(i * 16, 16)] = zeros16
        acc_c[pl.ds(i * 16, 16)] = zeros16

    def _start(c, b):
        off = base + c * CHUNK
        pltpu.async_copy(lg_hbm.at[pl.ds(off, CHUNK)], lg_v[b], sems[b])
        pltpu.async_copy(tg_hbm.at[pl.ds(off, CHUNK)], tg_v[b], sems[b])
        pltpu.async_copy(ei_hbm.at[pl.ds(off, CHUNK)], idx_v[b], sems[b])

    def _wait(c, b):
        off = base + c * CHUNK
        pltpu.make_async_copy(lg_hbm.at[pl.ds(off, CHUNK)], lg_v[b], sems[b]).wait()
        pltpu.make_async_copy(tg_hbm.at[pl.ds(off, CHUNK)], tg_v[b], sems[b]).wait()
        pltpu.make_async_copy(ei_hbm.at[pl.ds(off, CHUNK)], idx_v[b], sems[b]).wait()

    def _compute(b):
        @plsc.parallel_loop(0, VREGS_PER_CHUNK, unroll=5)
        def _inner(i):
            sl = pl.ds(i * 16, 16)
            src = idx_v[b][sl]
            g = plsc.load_gather(nb_v, [src])
            addr = lane64 + g
            l = lg_v[b][sl]
            t = tg_v[b][sl]
            u = jnp.exp(-jnp.abs(l))
            raw = jnp.maximum(l, 0.0) - l * t + _log1p_poly(u)
            plsc.addupdate_scatter(acc_l, [addr], raw)
            plsc.addupdate_scatter(acc_c, [addr], ones16)

    _start(0, 0)
    _start(1, 1)

    @pl.loop(0, (NCHUNK - 2) // 2)
    def _ring(gi):
        for b in range(2):
            c = gi * 2 + b
            _wait(c, b)
            _start(c + 2, b)
            _compute(b)

    for b in range(2):
        _wait(NCHUNK - 2 + b, b)
        _compute(b)

    half = N_BINS * 16 // 2
    while half >= N_BINS:
        for o in range(0, half, 16):
            sl_lo = pl.ds(o, 16)
            sl_hi = pl.ds(o + half, 16)
            acc_l[sl_lo] = acc_l[sl_lo] + acc_l[sl_hi]
            acc_c[sl_lo] = acc_c[sl_lo] + acc_c[sl_hi]
        half //= 2

    pltpu.sync_copy(acc_l.at[pl.ds(0, N_BINS)], outl_hbm.at[pl.ds(w * N_BINS, N_BINS)])
    pltpu.sync_copy(acc_c.at[pl.ds(0, N_BINS)], outc_hbm.at[pl.ds(w * N_BINS, N_BINS)])


_sc_call = pl.kernel(
    _sc_body,
    out_type=[
        jax.ShapeDtypeStruct((NW * N_BINS,), jnp.float32),
        jax.ShapeDtypeStruct((NW * N_BINS,), jnp.float32),
    ],
    mesh=plsc.VectorSubcoreMesh(core_axis_name="c", subcore_axis_name="s"),
    compiler_params=pltpu.CompilerParams(needs_layout_passes=False),
    scratch_types=[
        pltpu.VMEM((N_NODES,), jnp.int32),
        pltpu.VMEM((CHUNK,), jnp.int32),
        pltpu.VMEM((CHUNK,), jnp.int32),
        pltpu.VMEM((CHUNK,), jnp.float32),
        pltpu.VMEM((CHUNK,), jnp.float32),
        pltpu.VMEM((CHUNK,), jnp.float32),
        pltpu.VMEM((CHUNK,), jnp.float32),
        pltpu.VMEM((N_BINS * 16,), jnp.float32),
        pltpu.VMEM((N_BINS * 16,), jnp.float32),
        pltpu.SemaphoreType.DMA,
        pltpu.SemaphoreType.DMA,
    ],
)


def _fin_body(l_ref, c_ref, n_ref, o_ref):
    gl = jnp.sum(l_ref[...], axis=0, keepdims=True)
    gc = jnp.sum(c_ref[...], axis=0, keepdims=True)
    ratio = gl / jnp.maximum(gc, 1.0)
    o_ref[...] = jnp.sum(ratio, keepdims=True) / n_ref[...]


_fin_call = pl.pallas_call(
    _fin_body,
    out_shape=jax.ShapeDtypeStruct((1, 1), jnp.float32),
)


def kernel(logits, target, node_batch, edge_index):
    logits = logits.astype(jnp.float32)
    target = target.astype(jnp.float32)
    nb = node_batch.astype(jnp.int32)
    ei = edge_index.astype(jnp.int32).reshape(-1)
    outl, outc = _sc_call(logits, target, ei, nb)
    outl = outl.reshape(NW, N_BINS)
    outc = outc.reshape(NW, N_BINS)
    n_graphs = (nb[-1].astype(jnp.float32) + 1.0).reshape(1, 1)
    return _fin_call(outl, outc, n_graphs)[0, 0]


# speedup vs baseline: 451.8156x; 1.1328x over previous
import jax
import jax.numpy as jnp
from jax import lax
from jax.experimental import pallas as pl
from jax.experimental.pallas import tpu as pltpu
from jax.experimental.pallas import tpu_sc as plsc

N_NODES = 100000
N_EDGES = 3200000
N_BINS = 64

NUM_CORES = 2
NUM_SUBCORES = 16
NW = NUM_CORES * NUM_SUBCORES
EDGES_PER_TILE = N_EDGES // NW
CHUNK = 2000
NCHUNK = EDGES_PER_TILE // CHUNK
VREGS_PER_CHUNK = CHUNK // 16

_LOG1P_C = (
    1.6936626588279148e-06,
    0.9998325947817008,
    -0.4972033312209412,
    0.31504127991169084,
    -0.18901954822862585,
    0.08152317762235059,
    -0.017029610590701707,
)


def _log1p_poly(u):
    p = jnp.full((16,), _LOG1P_C[-1], dtype=jnp.float32)
    for c in _LOG1P_C[-2::-1]:
        p = p * u + jnp.float32(c)
    return p


def _sc_body(lg_hbm, tg_hbm, ei_hbm, nb_hbm, outl_hbm, outc_hbm,
             nb_v, idx_v0, idx_v1, lg_v0, lg_v1, tg_v0, tg_v1, acc_l, acc_c,
             sem0, sem1):
    w = lax.axis_index("c") * NUM_SUBCORES + lax.axis_index("s")
    base = w * EDGES_PER_TILE
    sems = (sem0, sem1)
    idx_v = (idx_v0, idx_v1)
    lg_v = (lg_v0, lg_v1)
    tg_v = (tg_v0, tg_v1)

    pltpu.sync_copy(nb_hbm, nb_v)

    zeros16 = jnp.zeros((16,), jnp.float32)
    ones16 = jnp.ones((16,), jnp.float32)
    lane64 = lax.iota(jnp.int32, 16) * N_BINS

    @pl.loop(0, N_BINS)
    def _zero(i):
        acc_l[pl.ds(i * 16, 16)] = zeros16
        acc_c[pl.ds(i * 16, 16)] = zeros16

    def _start(c, b):
        off = base + c * CHUNK
        pltpu.async_copy(lg_hbm.at[pl.ds(off, CHUNK)], lg_v[b], sems[b])
        pltpu.async_copy(tg_hbm.at[pl.ds(off, CHUNK)], tg_v[b], sems[b])
        pltpu.async_copy(ei_hbm.at[pl.ds(off, CHUNK)], idx_v[b], sems[b])

    def _wait(c, b):
        off = base + c * CHUNK
        pltpu.make_async_copy(lg_hbm.at[pl.ds(off, CHUNK)], lg_v[b], sems[b]).wait()
        pltpu.make_async_copy(tg_hbm.at[pl.ds(off, CHUNK)], tg_v[b], sems[b]).wait()
        pltpu.make_async_copy(ei_hbm.at[pl.ds(off, CHUNK)], idx_v[b], sems[b]).wait()

    def _compute(b):
        @plsc.parallel_loop(0, VREGS_PER_CHUNK, unroll=5)
        def _inner(i):
            sl = pl.ds(i * 16, 16)
            src = idx_v[b][sl]
            g = plsc.load_gather(nb_v, [src])
            addr = lane64 + g
            l = lg_v[b][sl]
            t = tg_v[b][sl]
            u = jnp.exp(-jnp.abs(l))
            raw = jnp.maximum(l, 0.0) - l * t + _log1p_poly(u)
            plsc.addupdate_scatter(acc_l, [addr], raw)
            plsc.addupdate_scatter(acc_c, [addr], ones16)

    _start(0, 0)
    _start(1, 1)

    @pl.loop(0, (NCHUNK - 2) // 2)
    def _ring(gi):
        for b in range(2):
            c = gi * 2 + b
            _wait(c, b)
            _start(c + 2, b)
            _compute(b)

    for b in range(2):
        _wait(NCHUNK - 2 + b, b)
        _compute(b)

    half = N_BINS * 16 // 2
    while half >= N_BINS:
        for o in range(0, half, 16):
            sl_lo = pl.ds(o, 16)
            sl_hi = pl.ds(o + half, 16)
            acc_l[sl_lo] = acc_l[sl_lo] + acc_l[sl_hi]
            acc_c[sl_lo] = acc_c[sl_lo] + acc_c[sl_hi]
        half //= 2

    pltpu.sync_copy(acc_l.at[pl.ds(0, N_BINS)], outl_hbm.at[pl.ds(w * N_BINS, N_BINS)])
    pltpu.sync_copy(acc_c.at[pl.ds(0, N_BINS)], outc_hbm.at[pl.ds(w * N_BINS, N_BINS)])


_sc_call = pl.kernel(
    _sc_body,
    out_type=[
        jax.ShapeDtypeStruct((NW * N_BINS,), jnp.float32),
        jax.ShapeDtypeStruct((NW * N_BINS,), jnp.float32),
    ],
    mesh=plsc.VectorSubcoreMesh(core_axis_name="c", subcore_axis_name="s"),
    compiler_params=pltpu.CompilerParams(needs_layout_passes=False),
    scratch_types=[
        pltpu.VMEM((N_NODES,), jnp.int32),
        pltpu.VMEM((CHUNK,), jnp.int32),
        pltpu.VMEM((CHUNK,), jnp.int32),
        pltpu.VMEM((CHUNK,), jnp.float32),
        pltpu.VMEM((CHUNK,), jnp.float32),
        pltpu.VMEM((CHUNK,), jnp.float32),
        pltpu.VMEM((CHUNK,), jnp.float32),
        pltpu.VMEM((N_BINS * 16,), jnp.float32),
        pltpu.VMEM((N_BINS * 16,), jnp.float32),
        pltpu.SemaphoreType.DMA,
        pltpu.SemaphoreType.DMA,
    ],
)


def _fin_body(l_ref, c_ref, n_ref, o_ref):
    gl = jnp.sum(l_ref[...], axis=0, keepdims=True)
    gc = jnp.sum(c_ref[...], axis=0, keepdims=True)
    ratio = gl / jnp.maximum(gc, 1.0)
    o_ref[...] = jnp.sum(ratio, keepdims=True) / n_ref[...]


_fin_call = pl.pallas_call(
    _fin_body,
    out_shape=jax.ShapeDtypeStruct((1, 1), jnp.float32),
)


def kernel(logits, target, node_batch, edge_index):
    logits = logits.astype(jnp.float32)
    target = target.astype(jnp.float32)
    nb = node_batch.astype(jnp.int32)
    ei = edge_index[0].astype(jnp.int32)
    outl, outc = _sc_call(logits, target, ei, nb)
    outl = outl.reshape(NW, N_BINS)
    outc = outc.reshape(NW, N_BINS)
    n_graphs = (nb[-1].astype(jnp.float32) + 1.0).reshape(1, 1)
    return _fin_call(outl, outc, n_graphs)[0, 0]
